# baseline (speedup 1.0000x reference)
import functools

import jax
import jax.numpy as jnp
from jax import lax
from jax.experimental import pallas as pl
from jax.experimental.pallas import tpu as pltpu
from jax.experimental.pallas import tpu_sc as plsc

NC = 2
NS = 16
NW = NC * NS
CHUNK = 128
RING = 3
AHEAD = 2


@functools.lru_cache(maxsize=None)
def _make_gather(N, D):
    b_per_w = N // NW
    nchunks = b_per_w // CHUNK
    mesh = plsc.VectorSubcoreMesh(core_axis_name="c", subcore_axis_name="s")

    @functools.partial(
        pl.kernel,
        mesh=mesh,
        out_type=[
            jax.ShapeDtypeStruct((N, D), jnp.float32),
            jax.ShapeDtypeStruct((N, D), jnp.float32),
        ],
        scratch_types=[
            pltpu.VMEM((nchunks, CHUNK), jnp.int32),
            pltpu.VMEM((RING, CHUNK, 2, D), jnp.float32),
        ] + [pltpu.SemaphoreType.DMA] * (2 * RING),
    )
    def k(idx_hbm, tab_hbm, cos_out, sin_out, idx_v, buf, *sems):
        wid = lax.axis_index("s") * NC + lax.axis_index("c")
        base = wid * b_per_w
        gsems = sems[:RING]
        ssems = sems[RING:]
        pltpu.sync_copy(idx_hbm.at[wid], idx_v)

        def fire_gather(j):
            s = j % RING
            return (
                pltpu.async_copy(tab_hbm.at[idx_v.at[j]], buf.at[s], gsems[s]),
            )

        def fire_scatter(j):
            s = j % RING
            row0 = base + j * CHUNK
            return (
                pltpu.async_copy(buf.at[s, :, 0], cos_out.at[pl.ds(row0, CHUNK)], ssems[s]),
                pltpu.async_copy(buf.at[s, :, 1], sin_out.at[pl.ds(row0, CHUNK)], ssems[s]),
            )

        gd = [None] * RING
        pend = [None] * RING
        for j in range(min(AHEAD, nchunks)):
            gd[j % RING] = fire_gather(j)
        for j in range(nchunks):
            s = j % RING
            for d in gd[s]:
                d.wait()
            pend[s] = fire_scatter(j)
            nxt = j + AHEAD
            if nxt < nchunks:
                o = nxt % RING
                if pend[o] is not None:
                    for d in pend[o]:
                        d.wait()
                    pend[o] = None
                gd[o] = fire_gather(nxt)
        for p in pend:
            if p is not None:
                for d in p:
                    d.wait()

    return k


def kernel(positions, cos_cached, sin_cached):
    B, S = positions.shape
    V, D = cos_cached.shape
    N = B * S
    idx = positions.astype(jnp.int32).reshape(NW, N // (NW * CHUNK), CHUNK)
    fused = jnp.stack([cos_cached, sin_cached], axis=1)
    cos, sin = _make_gather(N, D)(idx, fused)
    return cos.reshape(B, S, D), sin.reshape(B, S, D)


# speedup vs baseline: 1.0839x; 1.0839x over previous
import functools

import jax
import jax.numpy as jnp
from jax import lax
from jax.experimental import pallas as pl
from jax.experimental.pallas import tpu as pltpu
from jax.experimental.pallas import tpu_sc as plsc

NC = 2
NS = 16
NW = NC * NS
CHUNK = 128
RING = 3
AHEAD = 2


@functools.lru_cache(maxsize=None)
def _make_gather(N, D):
    b_per_w = N // NW
    nchunks = b_per_w // CHUNK
    half = D // 2
    mesh = plsc.VectorSubcoreMesh(core_axis_name="c", subcore_axis_name="s")

    @functools.partial(
        pl.kernel,
        mesh=mesh,
        out_type=[
            jax.ShapeDtypeStruct((N, D), jnp.float32),
            jax.ShapeDtypeStruct((N, D), jnp.float32),
        ],
        scratch_types=[
            pltpu.VMEM((nchunks, CHUNK), jnp.int32),
            pltpu.VMEM((RING, CHUNK, D), jnp.float32),
        ] + [pltpu.SemaphoreType.DMA] * (2 * RING),
        compiler_params=pltpu.CompilerParams(use_tc_tiling_on_sc=False),
    )
    def k(idx_hbm, tab_hbm, cos_out, sin_out, idx_v, buf, *sems):
        wid = lax.axis_index("s") * NC + lax.axis_index("c")
        base = wid * b_per_w
        gsems = sems[:RING]
        ssems = sems[RING:]
        pltpu.sync_copy(idx_hbm.at[wid], idx_v)

        def fire_gather(j):
            s = j % RING
            return (
                pltpu.async_copy(tab_hbm.at[idx_v.at[j]], buf.at[s], gsems[s]),
            )

        def fire_scatter(j):
            s = j % RING
            rows = pl.ds(base + j * CHUNK, CHUNK)
            c_half = buf.at[s, :, pl.ds(0, half)]
            s_half = buf.at[s, :, pl.ds(half, half)]
            return (
                pltpu.async_copy(c_half, cos_out.at[rows, pl.ds(0, half)], ssems[s]),
                pltpu.async_copy(c_half, cos_out.at[rows, pl.ds(half, half)], ssems[s]),
                pltpu.async_copy(s_half, sin_out.at[rows, pl.ds(0, half)], ssems[s]),
                pltpu.async_copy(s_half, sin_out.at[rows, pl.ds(half, half)], ssems[s]),
            )

        gd = [None] * RING
        pend = [None] * RING
        for j in range(min(AHEAD, nchunks)):
            gd[j % RING] = fire_gather(j)
        for j in range(nchunks):
            s = j % RING
            for d in gd[s]:
                d.wait()
            pend[s] = fire_scatter(j)
            nxt = j + AHEAD
            if nxt < nchunks:
                o = nxt % RING
                if pend[o] is not None:
                    for d in pend[o]:
                        d.wait()
                    pend[o] = None
                gd[o] = fire_gather(nxt)
        for p in pend:
            if p is not None:
                for d in p:
                    d.wait()

    return k


def kernel(positions, cos_cached, sin_cached):
    B, S = positions.shape
    V, D = cos_cached.shape
    N = B * S
    half = D // 2
    idx = positions.astype(jnp.int32).reshape(NW, N // (NW * CHUNK), CHUNK)
    fused = jnp.concatenate(
        [cos_cached[:, :half], sin_cached[:, :half]], axis=1)
    cos, sin = _make_gather(N, D)(idx, fused)
    return cos.reshape(B, S, D), sin.reshape(B, S, D)


# speedup vs baseline: 1.1284x; 1.0411x over previous
import functools

import jax
import jax.numpy as jnp
from jax import lax
from jax.experimental import pallas as pl
from jax.experimental.pallas import tpu as pltpu
from jax.experimental.pallas import tpu_sc as plsc

NC = 2
NS = 16
NW = NC * NS
L = 16
CHUNK = 128
GRING = 3
AHEAD = 2
ORING = 2


@functools.lru_cache(maxsize=None)
def _make_gather(N, D):
    b_per_w = N // NW
    nchunks = b_per_w // CHUNK
    half = D // 2
    mesh = plsc.VectorSubcoreMesh(core_axis_name="c", subcore_axis_name="s")

    @functools.partial(
        pl.kernel,
        mesh=mesh,
        out_type=[
            jax.ShapeDtypeStruct((N, D), jnp.float32),
            jax.ShapeDtypeStruct((N, D), jnp.float32),
        ],
        scratch_types=[
            pltpu.VMEM((nchunks, CHUNK), jnp.int32),
            pltpu.VMEM((GRING, CHUNK, D), jnp.float32),
            pltpu.VMEM((ORING, CHUNK, D), jnp.float32),
            pltpu.VMEM((ORING, CHUNK, D), jnp.float32),
        ] + [pltpu.SemaphoreType.DMA] * (GRING + ORING),
        compiler_params=pltpu.CompilerParams(use_tc_tiling_on_sc=False),
    )
    def k(idx_hbm, tab_hbm, cos_out, sin_out, idx_v, fbuf, cbuf, sbuf, *sems):
        wid = lax.axis_index("s") * NC + lax.axis_index("c")
        base = wid * b_per_w
        gsems = sems[:GRING]
        ssems = sems[GRING:]
        pltpu.sync_copy(idx_hbm.at[wid], idx_v)

        def fire_gather(j):
            g = j % GRING
            return pltpu.async_copy(tab_hbm.at[idx_v.at[j]], fbuf.at[g], gsems[g])

        def expand(j):
            g = j % GRING
            o = j % ORING

            def body(r, _):
                for v in range(half // L):
                    cl = pl.ds(v * L, L)
                    cr = pl.ds(half + v * L, L)
                    c = fbuf[g, r, cl]
                    s = fbuf[g, r, cr]
                    cbuf[o, r, cl] = c
                    cbuf[o, r, cr] = c
                    sbuf[o, r, cl] = s
                    sbuf[o, r, cr] = s
                return _

            lax.fori_loop(0, CHUNK, body, 0, unroll=4)

        def fire_scatter(j):
            o = j % ORING
            rows = pl.ds(base + j * CHUNK, CHUNK)
            return (
                pltpu.async_copy(cbuf.at[o], cos_out.at[rows], ssems[o]),
                pltpu.async_copy(sbuf.at[o], sin_out.at[rows], ssems[o]),
            )

        gd = [None] * GRING
        pend = [None] * ORING
        for j in range(min(AHEAD, nchunks)):
            gd[j % GRING] = fire_gather(j)
        for j in range(nchunks):
            g = j % GRING
            o = j % ORING
            gd[g].wait()
            if pend[o] is not None:
                for d in pend[o]:
                    d.wait()
                pend[o] = None
            expand(j)
            pend[o] = fire_scatter(j)
            nxt = j + AHEAD
            if nxt < nchunks:
                gd[nxt % GRING] = fire_gather(nxt)
        for p in pend:
            if p is not None:
                for d in p:
                    d.wait()

    return k


def kernel(positions, cos_cached, sin_cached):
    B, S = positions.shape
    V, D = cos_cached.shape
    N = B * S
    half = D // 2
    idx = positions.astype(jnp.int32).reshape(NW, N // (NW * CHUNK), CHUNK)
    fused = jnp.concatenate(
        [cos_cached[:, :half], sin_cached[:, :half]], axis=1)
    cos, sin = _make_gather(N, D)(idx, fused)
    return cos.reshape(B, S, D), sin.reshape(B, S, D)


# speedup vs baseline: 1.1756x; 1.0418x over previous
import functools

import jax
import jax.numpy as jnp
from jax import lax
from jax.experimental import pallas as pl
from jax.experimental.pallas import tpu as pltpu
from jax.experimental.pallas import tpu_sc as plsc

NC = 2
NS = 16
NW = NC * NS
L = 16
CHUNK = 128
RING = 3
AHEAD = 2


@functools.lru_cache(maxsize=None)
def _make_gather(N, D):
    b_per_w = N // NW
    nchunks = b_per_w // CHUNK
    half = D // 2
    mesh = plsc.VectorSubcoreMesh(core_axis_name="c", subcore_axis_name="s")

    @functools.partial(
        pl.kernel,
        mesh=mesh,
        out_type=[
            jax.ShapeDtypeStruct((N, D), jnp.float32),
            jax.ShapeDtypeStruct((N, D), jnp.float32),
        ],
        scratch_types=[
            pltpu.VMEM((nchunks, CHUNK), jnp.int32),
            pltpu.VMEM((RING, CHUNK, D), jnp.float32),
            pltpu.VMEM((RING, CHUNK, D), jnp.float32),
        ] + [pltpu.SemaphoreType.DMA] * (2 * RING),
        compiler_params=pltpu.CompilerParams(use_tc_tiling_on_sc=False),
    )
    def k(idx_hbm, tab_hbm, cos_out, sin_out, idx_v, cbuf, sbuf, *sems):
        wid = lax.axis_index("s") * NC + lax.axis_index("c")
        base = wid * b_per_w
        gsems = sems[:RING]
        ssems = sems[RING:]
        pltpu.sync_copy(idx_hbm.at[wid], idx_v)

        def fire_gather(j):
            g = j % RING
            return pltpu.async_copy(tab_hbm.at[idx_v.at[j]], cbuf.at[g], gsems[g])

        def expand(j):
            g = j % RING

            @plsc.parallel_loop(0, CHUNK, unroll=4)
            def _(r):
                for v in range(half // L):
                    cl = pl.ds(v * L, L)
                    cr = pl.ds(half + v * L, L)
                    c = cbuf[g, r, cl]
                    s = cbuf[g, r, cr]
                    sbuf[g, r, cl] = s
                    sbuf[g, r, cr] = s
                    cbuf[g, r, cr] = c

        def fire_scatter(j):
            g = j % RING
            rows = pl.ds(base + j * CHUNK, CHUNK)
            return (
                pltpu.async_copy(cbuf.at[g], cos_out.at[rows], ssems[g]),
                pltpu.async_copy(sbuf.at[g], sin_out.at[rows], ssems[g]),
            )

        gd = [None] * RING
        pend = [None] * RING
        for j in range(min(AHEAD, nchunks)):
            gd[j % RING] = fire_gather(j)
        for j in range(nchunks):
            g = j % RING
            gd[g].wait()
            expand(j)
            pend[g] = fire_scatter(j)
            nxt = j + AHEAD
            if nxt < nchunks:
                o = nxt % RING
                if pend[o] is not None:
                    for d in pend[o]:
                        d.wait()
                    pend[o] = None
                gd[o] = fire_gather(nxt)
        for p in pend:
            if p is not None:
                for d in p:
                    d.wait()

    return k


def kernel(positions, cos_cached, sin_cached):
    B, S = positions.shape
    V, D = cos_cached.shape
    N = B * S
    half = D // 2
    idx = positions.astype(jnp.int32).reshape(NW, N // (NW * CHUNK), CHUNK)
    fused = jnp.concatenate(
        [cos_cached[:, :half], sin_cached[:, :half]], axis=1)
    cos, sin = _make_gather(N, D)(idx, fused)
    return cos.reshape(B, S, D), sin.reshape(B, S, D)
